# baseline (device time: 421201 ns/iter reference)
import jax
import jax.numpy as jnp
from jax import lax
from jax.experimental import pallas as pl
from jax.experimental.pallas import tpu as pltpu

N_DEV = 4
TC = 256


def kernel(x, A, B, C):
    Bb, S, D = x.shape
    N = B.shape[-1]

    def body(x_ref, a_ref, b_ref, c_ref, y_ref,
             send_buf, recv_buf, send_sem, recv_sem):
        my = lax.axis_index("i")
        left = (my - 1) % N_DEV
        right = (my + 1) % N_DEV

        da = jnp.exp(a_ref[...]).T[None]

        def step(t, h):
            xt = x_ref[:, pl.ds(t, 1), :]
            btv = jnp.transpose(b_ref[:, pl.ds(t, 1), :], (0, 2, 1))
            ctv = jnp.transpose(c_ref[:, pl.ds(t, 1), :], (0, 2, 1))
            h = h * da + xt * btv
            y_ref[:, pl.ds(t, 1), :] = jnp.sum(h * ctv, axis=1, keepdims=True)
            return h

        h0 = jnp.zeros((Bb, N, D), jnp.float32)
        h_final = lax.fori_loop(0, S, step, h0)
        send_buf[...] = h_final

        cp = pltpu.make_async_remote_copy(
            src_ref=send_buf,
            dst_ref=recv_buf,
            send_sem=send_sem,
            recv_sem=recv_sem,
            device_id=(right,),
            device_id_type=pl.DeviceIdType.MESH,
        )
        del left

        @pl.when(my > 0)
        def _():
            cp.wait_recv()
            pow_s = jnp.exp(a_ref[...] * float(S)).T[None]
            send_buf[...] = pow_s * recv_buf[...] + send_buf[...]

        @pl.when(my < N_DEV - 1)
        def _():
            cp.start()

        @pl.when(my > 0)
        def _():
            lax.fori_loop(0, TC, step, recv_buf[...])

        @pl.when(my < N_DEV - 1)
        def _():
            cp.wait_send()

    return pl.pallas_call(
        body,
        out_shape=jax.ShapeDtypeStruct((Bb, S, D), jnp.float32),
        in_specs=[pl.BlockSpec(memory_space=pltpu.VMEM)] * 4,
        out_specs=pl.BlockSpec(memory_space=pltpu.VMEM),
        scratch_shapes=[
            pltpu.VMEM((Bb, N, D), jnp.float32),
            pltpu.VMEM((Bb, N, D), jnp.float32),
            pltpu.SemaphoreType.DMA,
            pltpu.SemaphoreType.DMA,
        ],
    )(x, A, B, C)


# device time: 293131 ns/iter; 1.4369x vs baseline; 1.4369x over previous
import jax
import jax.numpy as jnp
from jax import lax
from jax.experimental import pallas as pl
from jax.experimental.pallas import tpu as pltpu

N_DEV = 4
TC = 256


def kernel(x, A, B, C):
    Bb, S, D = x.shape
    N = B.shape[-1]

    def body(x_ref, a_ref, b_ref, c_ref, y_ref,
             send_buf, recv_buf, send_sem, recv_sem):
        my = lax.axis_index("i")
        left = (my - 1) % N_DEV
        right = (my + 1) % N_DEV

        da = jnp.exp(a_ref[...]).T[None]

        def step(t, h):
            xt = x_ref[:, pl.ds(t, 1), :]
            btv = jnp.transpose(b_ref[:, pl.ds(t, 1), :], (0, 2, 1))
            ctv = jnp.transpose(c_ref[:, pl.ds(t, 1), :], (0, 2, 1))
            h = h * da + xt * btv
            y_ref[:, pl.ds(t, 1), :] = jnp.sum(h * ctv, axis=1, keepdims=True)
            return h

        h0 = jnp.zeros((Bb, N, D), jnp.float32)
        h_final = lax.fori_loop(0, S, step, h0, unroll=8)
        send_buf[...] = h_final

        cp = pltpu.make_async_remote_copy(
            src_ref=send_buf,
            dst_ref=recv_buf,
            send_sem=send_sem,
            recv_sem=recv_sem,
            device_id=(right,),
            device_id_type=pl.DeviceIdType.MESH,
        )
        del left

        @pl.when(my > 0)
        def _():
            cp.wait_recv()
            pow_s = jnp.exp(a_ref[...] * float(S)).T[None]
            send_buf[...] = pow_s * recv_buf[...] + send_buf[...]

        @pl.when(my < N_DEV - 1)
        def _():
            cp.start()

        @pl.when(my > 0)
        def _():
            lax.fori_loop(0, TC, step, recv_buf[...], unroll=8)

        @pl.when(my < N_DEV - 1)
        def _():
            cp.wait_send()

    return pl.pallas_call(
        body,
        out_shape=jax.ShapeDtypeStruct((Bb, S, D), jnp.float32),
        in_specs=[pl.BlockSpec(memory_space=pltpu.VMEM)] * 4,
        out_specs=pl.BlockSpec(memory_space=pltpu.VMEM),
        scratch_shapes=[
            pltpu.VMEM((Bb, N, D), jnp.float32),
            pltpu.VMEM((Bb, N, D), jnp.float32),
            pltpu.SemaphoreType.DMA,
            pltpu.SemaphoreType.DMA,
        ],
    )(x, A, B, C)
